# Initial kernel scaffold; baseline (speedup 1.0000x reference)
import jax
import jax.numpy as jnp
from jax.experimental import pallas as pl


def kernel(annotations, ids):
    raise NotImplementedError("write your pallas kernel here")


# speedup vs baseline: 2.7593x; 2.7593x over previous
import functools

import jax
import jax.numpy as jnp
from jax import lax
from jax.experimental import pallas as pl
from jax.experimental.pallas import tpu as pltpu
from jax.experimental.pallas import tpu_sc as plsc

NC, NS = 2, 16
NW = NC * NS
CHUNK = 128


def _make_gather(total: int, dim: int):
    ids_per_w = total // NW
    nchunk = ids_per_w // CHUNK

    def body(table_hbm, idx_hbm, out_hbm, idx_v, rows_v, sem):
        wid = lax.axis_index("s") * NC + lax.axis_index("c")
        base = wid * ids_per_w

        @pl.loop(0, nchunk)
        def _(ci):
            off = base + ci * CHUNK
            pltpu.sync_copy(idx_hbm.at[pl.ds(off, CHUNK)], idx_v)
            pltpu.async_copy(table_hbm.at[idx_v], rows_v, sem).wait()
            pltpu.sync_copy(rows_v, out_hbm.at[pl.ds(off, CHUNK)])

    return pl.kernel(
        body,
        out_type=jax.ShapeDtypeStruct((total, dim), jnp.float32),
        mesh=plsc.VectorSubcoreMesh(core_axis_name="c", subcore_axis_name="s"),
        scratch_types=[
            pltpu.VMEM((CHUNK,), jnp.int32),
            pltpu.VMEM((CHUNK, dim), jnp.float32),
            pltpu.SemaphoreType.DMA,
        ],
    )


def kernel(annotations, ids):
    batch, hist = ids.shape
    vocab, dim = annotations.shape
    total = batch * hist
    flat = ids.reshape(total).astype(jnp.int32)

    grain = NW * CHUNK
    padded = (total + grain - 1) // grain * grain
    if padded != total:
        flat = jnp.pad(flat, (0, padded - total))

    out = _make_gather(padded, dim)(annotations, flat)
    return out[:total].reshape(batch, hist, dim)


# speedup vs baseline: 3.3572x; 1.2167x over previous
import jax
import jax.numpy as jnp
from jax import lax
from jax.experimental import pallas as pl
from jax.experimental.pallas import tpu as pltpu
from jax.experimental.pallas import tpu_sc as plsc

NC, NS = 2, 16
NW = NC * NS
CHUNK = 128
NBUF = 5
GAHEAD = 3


def _make_gather(total: int, dim: int):
    ids_per_w = total // NW
    nchunk = ids_per_w // CHUNK
    nouter = nchunk // NBUF

    def body(table_hbm, idx_hbm, out_hbm, idx_v, rows_v, gsem, osem):
        wid = lax.axis_index("s") * NC + lax.axis_index("c")
        base = wid * ids_per_w

        def out_at(ci):
            return out_hbm.at[pl.ds(base + ci * CHUNK, CHUNK)]

        pltpu.sync_copy(idx_hbm.at[pl.ds(base, ids_per_w)], idx_v)

        def idx_at(ci):
            return idx_v.at[pl.ds(ci * CHUNK, CHUNK)]

        def start_gather(ci, buf):
            pltpu.async_copy(table_hbm.at[idx_at(ci)], rows_v.at[buf], gsem)

        for b in range(GAHEAD):
            start_gather(b, b)

        @pl.loop(0, nouter)
        def _(gi):
            for b in range(NBUF):
                ci = gi * NBUF + b
                nb = (b + GAHEAD) % NBUF

                def wait_store(pci=ci + GAHEAD - NBUF, pb=nb):
                    pltpu.make_async_copy(rows_v.at[pb], out_at(pci), osem).wait()

                if b < NBUF - GAHEAD:
                    @pl.when(gi > 0)
                    def _():
                        wait_store()
                else:
                    wait_store()

                if b < NBUF - GAHEAD:
                    start_gather(ci + GAHEAD, nb)
                else:
                    @pl.when(gi < nouter - 1)
                    def _():
                        start_gather(ci + GAHEAD, nb)

                pltpu.make_async_copy(
                    table_hbm.at[idx_at(ci)], rows_v.at[b], gsem
                ).wait()
                pltpu.async_copy(rows_v.at[b], out_at(ci), osem)

        for b in range(GAHEAD, NBUF):
            ci = (nouter - 1) * NBUF + b
            pltpu.make_async_copy(rows_v.at[b], out_at(ci), osem).wait()

    return pl.kernel(
        body,
        out_type=jax.ShapeDtypeStruct((total, dim), jnp.float32),
        mesh=plsc.VectorSubcoreMesh(core_axis_name="c", subcore_axis_name="s"),
        scratch_types=[
            pltpu.VMEM((ids_per_w,), jnp.int32),
            pltpu.VMEM((NBUF, CHUNK, dim), jnp.float32),
            pltpu.SemaphoreType.DMA,
            pltpu.SemaphoreType.DMA,
        ],
    )


def kernel(annotations, ids):
    batch, hist = ids.shape
    vocab, dim = annotations.shape
    total = batch * hist
    flat = ids.reshape(total).astype(jnp.int32)

    grain = NW * CHUNK * NBUF
    padded = (total + grain - 1) // grain * grain
    if padded != total:
        flat = jnp.pad(flat, (0, padded - total))

    out = _make_gather(padded, dim)(annotations, flat)
    return out[:total].reshape(batch, hist, dim)
